# baseline (speedup 1.0000x reference)
import functools

import jax
import jax.numpy as jnp
from jax.experimental import pallas as pl
from jax.experimental.pallas import tpu as pltpu

_NB = 4096
_MC = 512
_K = 8


def _fitter_body(rot_ref, trans_ref, src_ref, tgt_ref, out_ref, dist_ref,
                 lsum_ref, ymat_ref):
    M = tgt_ref.shape[1]
    f32 = jnp.float32

    def _build_ymat():
        y0 = tgt_ref[0, :, 0:1]
        y1 = tgt_ref[0, :, 1:2]
        y2 = tgt_ref[0, :, 2:3]
        yn = y0 * y0 + y1 * y1 + y2 * y2
        ynh = yn.astype(jnp.bfloat16).astype(f32)
        ynl = yn - ynh
        zeros = jnp.zeros((M, _K - 5), dtype=f32)
        ymat = jnp.concatenate([y0, y1, y2, ynh, ynl, zeros], axis=1)
        ymat_ref[...] = ymat.astype(jnp.bfloat16)

    _build_ymat()

    rx = rot_ref[0:1, 0:1]
    ry = rot_ref[0:1, 1:2]
    rz = rot_ref[0:1, 2:3]
    t0 = trans_ref[0:1, 0:1]
    t1 = trans_ref[0:1, 1:2]
    t2 = trans_ref[0:1, 2:3]

    nrm2 = jnp.clip(rx * rx + ry * ry + rz * rz, 1e-4, None)
    ang = jnp.sqrt(nrm2)
    inv = 1.0 / ang
    fac1 = inv * jnp.sin(ang)
    fac2 = inv * inv * (1.0 - jnp.cos(ang))
    xx = rx * rx
    yy = ry * ry
    zz = rz * rz
    xy = rx * ry
    xz = rx * rz
    yz = ry * rz
    r00 = 1.0 - fac2 * (yy + zz)
    r01 = fac2 * xy - fac1 * rz
    r02 = fac2 * xz + fac1 * ry
    r10 = fac2 * xy + fac1 * rz
    r11 = 1.0 - fac2 * (xx + zz)
    r12 = fac2 * yz - fac1 * rx
    r20 = fac2 * xz - fac1 * ry
    r21 = fac2 * yz + fac1 * rx
    r22 = 1.0 - fac2 * (xx + yy)

    def q(v):
        return v.astype(jnp.bfloat16).astype(f32)

    s0 = q(src_ref[0, 0:1, :])
    s1 = q(src_ref[0, 1:2, :])
    s2 = q(src_ref[0, 2:3, :])
    p0 = q(r00) * s0 + q(r01) * s1 + q(r02) * s2 + t0
    p1 = q(r10) * s0 + q(r11) * s1 + q(r12) * s2 + t1
    p2 = q(r20) * s0 + q(r21) * s1 + q(r22) * s2 + t2
    prows = jnp.concatenate([p0, p1, p2], axis=0)
    out_ref[0] = jnp.transpose(prows)

    ones = jnp.ones((1, _NB), dtype=f32)
    zrows = jnp.zeros((_K - 5, _NB), dtype=f32)
    xmat = jnp.concatenate([-2.0 * p0, -2.0 * p1, -2.0 * p2, ones, ones,
                            zrows], axis=0).astype(jnp.bfloat16)

    dn = (((1,), (0,)), ((), ()))
    mins = None
    for c in range(M // _MC):
        acc = jax.lax.dot_general(
            ymat_ref[c * _MC:(c + 1) * _MC, :], xmat, dn,
            preferred_element_type=f32)
        cmin = jnp.min(acc, axis=0, keepdims=True)
        mins = cmin if mins is None else jnp.minimum(mins, cmin)

    xn = p0 * p0 + p1 * p1 + p2 * p2
    dists = mins + xn
    dist_ref[0, 0:1, :] = dists
    lsum_ref[0, 0:1, 0:1] = jnp.sum(dists, axis=1, keepdims=True)


@functools.partial(jax.jit, static_argnums=())
def kernel(source_pcd, target_pcd, initial_rot, initial_trans):
    B, N, _ = source_pcd.shape
    M = target_pcd.shape[1]
    src_t = jnp.transpose(source_pcd, (0, 2, 1))
    rot2 = initial_rot.reshape(1, 3)
    trans2 = initial_trans.reshape(1, 3)
    out_t, _, lsums = pl.pallas_call(
        _fitter_body,
        grid=(B,),
        in_specs=[
            pl.BlockSpec((1, 3), lambda b: (0, 0)),
            pl.BlockSpec((1, 3), lambda b: (0, 0)),
            pl.BlockSpec((1, 3, _NB), lambda b: (b, 0, 0)),
            pl.BlockSpec((1, M, 3), lambda b: (b, 0, 0)),
        ],
        out_specs=[
            pl.BlockSpec((1, _NB, 3), lambda b: (b, 0, 0)),
            pl.BlockSpec((1, 1, _NB), lambda b: (b, 0, 0)),
            pl.BlockSpec((1, 1, 1), lambda b: (b, 0, 0)),
        ],
        out_shape=[
            jax.ShapeDtypeStruct((B, N, 3), jnp.float32),
            jax.ShapeDtypeStruct((B, 1, N), jnp.float32),
            jax.ShapeDtypeStruct((B, 1, 1), jnp.float32),
        ],
        scratch_shapes=[pltpu.VMEM((M, _K), jnp.bfloat16)],
        compiler_params=pltpu.CompilerParams(
            dimension_semantics=("parallel",)),
    )(rot2, trans2, src_t, target_pcd)
    loss = jnp.sum(lsums) / (B * N)
    return (out_t, loss)


# speedup vs baseline: 1.2376x; 1.2376x over previous
import functools

import jax
import jax.numpy as jnp
from jax.experimental import pallas as pl
from jax.experimental.pallas import tpu as pltpu

_NB = 4096
_MC = 512
_K = 8


def _fitter_body(rot_ref, trans_ref, src_ref, tgt_ref, out_ref, dist_ref,
                 lsum_ref, ymat_ref):
    M = tgt_ref.shape[2]
    f32 = jnp.float32

    def _build_ymat():
        y0 = tgt_ref[0, 0:1, :]
        y1 = tgt_ref[0, 1:2, :]
        y2 = tgt_ref[0, 2:3, :]
        yn = y0 * y0 + y1 * y1 + y2 * y2
        ynh = yn.astype(jnp.bfloat16).astype(f32)
        ynl = yn - ynh
        zeros = jnp.zeros((_K - 5, M), dtype=f32)
        ymat_t = jnp.concatenate([y0, y1, y2, ynh, ynl, zeros], axis=0)
        ymat_ref[...] = jnp.transpose(ymat_t).astype(jnp.bfloat16)

    _build_ymat()

    rx = rot_ref[0:1, 0:1]
    ry = rot_ref[0:1, 1:2]
    rz = rot_ref[0:1, 2:3]
    t0 = trans_ref[0:1, 0:1]
    t1 = trans_ref[0:1, 1:2]
    t2 = trans_ref[0:1, 2:3]

    nrm2 = jnp.clip(rx * rx + ry * ry + rz * rz, 1e-4, None)
    ang = jnp.sqrt(nrm2)
    inv = 1.0 / ang
    fac1 = inv * jnp.sin(ang)
    fac2 = inv * inv * (1.0 - jnp.cos(ang))
    xx = rx * rx
    yy = ry * ry
    zz = rz * rz
    xy = rx * ry
    xz = rx * rz
    yz = ry * rz
    r00 = 1.0 - fac2 * (yy + zz)
    r01 = fac2 * xy - fac1 * rz
    r02 = fac2 * xz + fac1 * ry
    r10 = fac2 * xy + fac1 * rz
    r11 = 1.0 - fac2 * (xx + zz)
    r12 = fac2 * yz - fac1 * rx
    r20 = fac2 * xz - fac1 * ry
    r21 = fac2 * yz + fac1 * rx
    r22 = 1.0 - fac2 * (xx + yy)

    def q(v):
        return v.astype(jnp.bfloat16).astype(f32)

    s0 = q(src_ref[0, 0:1, :])
    s1 = q(src_ref[0, 1:2, :])
    s2 = q(src_ref[0, 2:3, :])
    p0 = q(r00) * s0 + q(r01) * s1 + q(r02) * s2 + t0
    p1 = q(r10) * s0 + q(r11) * s1 + q(r12) * s2 + t1
    p2 = q(r20) * s0 + q(r21) * s1 + q(r22) * s2 + t2
    out_ref[0, 0:1, :] = p0
    out_ref[0, 1:2, :] = p1
    out_ref[0, 2:3, :] = p2

    ones = jnp.ones((1, _NB), dtype=f32)
    zrows = jnp.zeros((_K - 5, _NB), dtype=f32)
    xmat = jnp.concatenate([-2.0 * p0, -2.0 * p1, -2.0 * p2, ones, ones,
                            zrows], axis=0).astype(jnp.bfloat16)

    dn = (((1,), (0,)), ((), ()))
    mins = None
    for c in range(M // _MC):
        acc = jax.lax.dot_general(
            ymat_ref[c * _MC:(c + 1) * _MC, :], xmat, dn,
            preferred_element_type=f32)
        cmin = jnp.min(acc, axis=0, keepdims=True)
        mins = cmin if mins is None else jnp.minimum(mins, cmin)

    xn = p0 * p0 + p1 * p1 + p2 * p2
    dists = mins + xn
    dist_ref[0, 0:1, :] = dists
    lsum_ref[0, 0:1, 0:1] = jnp.sum(dists, axis=1, keepdims=True)


@functools.partial(jax.jit, static_argnums=())
def kernel(source_pcd, target_pcd, initial_rot, initial_trans):
    B, N, _ = source_pcd.shape
    M = target_pcd.shape[1]
    src_t = jnp.transpose(source_pcd, (0, 2, 1))
    tgt_t = jnp.transpose(target_pcd, (0, 2, 1))
    rot2 = initial_rot.reshape(1, 3)
    trans2 = initial_trans.reshape(1, 3)
    out_t, _, lsums = pl.pallas_call(
        _fitter_body,
        grid=(B,),
        in_specs=[
            pl.BlockSpec((1, 3), lambda b: (0, 0)),
            pl.BlockSpec((1, 3), lambda b: (0, 0)),
            pl.BlockSpec((1, 3, _NB), lambda b: (b, 0, 0)),
            pl.BlockSpec((1, 3, M), lambda b: (b, 0, 0)),
        ],
        out_specs=[
            pl.BlockSpec((1, 3, _NB), lambda b: (b, 0, 0)),
            pl.BlockSpec((1, 1, _NB), lambda b: (b, 0, 0)),
            pl.BlockSpec((1, 1, 1), lambda b: (b, 0, 0)),
        ],
        out_shape=[
            jax.ShapeDtypeStruct((B, 3, N), jnp.float32),
            jax.ShapeDtypeStruct((B, 1, N), jnp.float32),
            jax.ShapeDtypeStruct((B, 1, 1), jnp.float32),
        ],
        scratch_shapes=[pltpu.VMEM((M, _K), jnp.bfloat16)],
        compiler_params=pltpu.CompilerParams(
            dimension_semantics=("parallel",)),
    )(rot2, trans2, src_t, tgt_t)
    transformed = jnp.transpose(out_t, (0, 2, 1))
    loss = jnp.sum(lsums) / (B * N)
    return (transformed, loss)


# speedup vs baseline: 1.2388x; 1.0009x over previous
import functools

import jax
import jax.numpy as jnp
from jax.experimental import pallas as pl
from jax.experimental.pallas import tpu as pltpu

_NB = 4096
_MC = 512
_K = 8


def _fitter_body(rot_ref, trans_ref, src_ref, tgt_ref, out_ref, lsum_ref,
                 ymat_ref):
    M = tgt_ref.shape[2]
    f32 = jnp.float32

    def _build_ymat():
        y0 = tgt_ref[0, 0:1, :]
        y1 = tgt_ref[0, 1:2, :]
        y2 = tgt_ref[0, 2:3, :]
        yn = y0 * y0 + y1 * y1 + y2 * y2
        ynh = yn.astype(jnp.bfloat16).astype(f32)
        ynl = yn - ynh
        zeros = jnp.zeros((_K - 5, M), dtype=f32)
        ymat_t = jnp.concatenate([y0, y1, y2, ynh, ynl, zeros], axis=0)
        ymat_ref[...] = ymat_t.astype(jnp.bfloat16)

    _build_ymat()

    rx = rot_ref[0:1, 0:1]
    ry = rot_ref[0:1, 1:2]
    rz = rot_ref[0:1, 2:3]
    t0 = trans_ref[0:1, 0:1]
    t1 = trans_ref[0:1, 1:2]
    t2 = trans_ref[0:1, 2:3]

    nrm2 = jnp.clip(rx * rx + ry * ry + rz * rz, 1e-4, None)
    ang = jnp.sqrt(nrm2)
    inv = 1.0 / ang
    fac1 = inv * jnp.sin(ang)
    fac2 = inv * inv * (1.0 - jnp.cos(ang))
    xx = rx * rx
    yy = ry * ry
    zz = rz * rz
    xy = rx * ry
    xz = rx * rz
    yz = ry * rz
    r00 = 1.0 - fac2 * (yy + zz)
    r01 = fac2 * xy - fac1 * rz
    r02 = fac2 * xz + fac1 * ry
    r10 = fac2 * xy + fac1 * rz
    r11 = 1.0 - fac2 * (xx + zz)
    r12 = fac2 * yz - fac1 * rx
    r20 = fac2 * xz - fac1 * ry
    r21 = fac2 * yz + fac1 * rx
    r22 = 1.0 - fac2 * (xx + yy)

    def q(v):
        return v.astype(jnp.bfloat16).astype(f32)

    s0 = q(src_ref[0, 0:1, :])
    s1 = q(src_ref[0, 1:2, :])
    s2 = q(src_ref[0, 2:3, :])
    p0 = q(r00) * s0 + q(r01) * s1 + q(r02) * s2 + t0
    p1 = q(r10) * s0 + q(r11) * s1 + q(r12) * s2 + t1
    p2 = q(r20) * s0 + q(r21) * s1 + q(r22) * s2 + t2
    out_ref[0, 0:1, :] = p0
    out_ref[0, 1:2, :] = p1
    out_ref[0, 2:3, :] = p2

    ones = jnp.ones((1, _NB), dtype=f32)
    zrows = jnp.zeros((_K - 5, _NB), dtype=f32)
    xmat = jnp.concatenate([-2.0 * p0, -2.0 * p1, -2.0 * p2, ones, ones,
                            zrows], axis=0).astype(jnp.bfloat16)

    dn = (((0,), (0,)), ((), ()))
    mins = None
    for c in range(M // _MC):
        acc = jax.lax.dot_general(
            ymat_ref[:, c * _MC:(c + 1) * _MC], xmat, dn,
            preferred_element_type=f32)
        cmin = jnp.min(acc, axis=0, keepdims=True)
        mins = cmin if mins is None else jnp.minimum(mins, cmin)

    xn = p0 * p0 + p1 * p1 + p2 * p2
    dists = mins + xn
    lsum_ref[0, 0:1, 0:1] = jnp.sum(dists, axis=1, keepdims=True)


@functools.partial(jax.jit, static_argnums=())
def kernel(source_pcd, target_pcd, initial_rot, initial_trans):
    B, N, _ = source_pcd.shape
    M = target_pcd.shape[1]
    src_t = jnp.transpose(source_pcd, (0, 2, 1))
    tgt_t = jnp.transpose(target_pcd, (0, 2, 1))
    rot2 = initial_rot.reshape(1, 3)
    trans2 = initial_trans.reshape(1, 3)
    out_t, lsums = pl.pallas_call(
        _fitter_body,
        grid=(B,),
        in_specs=[
            pl.BlockSpec((1, 3), lambda b: (0, 0)),
            pl.BlockSpec((1, 3), lambda b: (0, 0)),
            pl.BlockSpec((1, 3, _NB), lambda b: (b, 0, 0)),
            pl.BlockSpec((1, 3, M), lambda b: (b, 0, 0)),
        ],
        out_specs=[
            pl.BlockSpec((1, 3, _NB), lambda b: (b, 0, 0)),
            pl.BlockSpec((1, 1, 1), lambda b: (b, 0, 0)),
        ],
        out_shape=[
            jax.ShapeDtypeStruct((B, 3, N), jnp.float32),
            jax.ShapeDtypeStruct((B, 1, 1), jnp.float32),
        ],
        scratch_shapes=[pltpu.VMEM((_K, M), jnp.bfloat16)],
        compiler_params=pltpu.CompilerParams(
            dimension_semantics=("parallel",)),
    )(rot2, trans2, src_t, tgt_t)
    transformed = jnp.transpose(out_t, (0, 2, 1))
    loss = jnp.sum(lsums) / (B * N)
    return (transformed, loss)
